# baseline (speedup 1.0000x reference)
import jax
import jax.numpy as jnp
from jax import lax
from jax.experimental import pallas as pl
from jax.experimental.pallas import tpu as pltpu

H0, W0, C_IN = 16, 16, 200
H1, W1 = 12, 12
H2, W2 = 10, 10
H3, W3 = 8, 8
C = 160
OUT_H, OUT_W = 512, 512

S_ROWS = 264
A1_ROWS = 152
A2_ROWS = 104


def _leaky(a, slope=0.2):
    return jnp.maximum(a, slope * a)


def _gelu(a):
    return 0.5 * a * (1.0 + lax.erf(a * 0.7071067811865476))


def _feat_kernel(s_ref, x_ref,
                 w1_ref, b1_ref, w2_ref, b2_ref, w3_ref, b3_ref,
                 w4_ref, b4_ref, w5_ref, b5_ref,
                 wms_ref, wmx_ref, bm_ref, wcs_ref, wcx_ref, bc_ref,
                 out_ref,
                 s_scr, wide_ref, a1_ref, a2_ref, a3_ref):
    f32 = jnp.float32
    bf16 = jnp.bfloat16
    B = s_ref.shape[0]

    s_scr[:, :H0 * W0, :] = jnp.swapaxes(s_ref[...].astype(bf16), 1, 2)
    s_scr[:, H0 * W0:, :] = jnp.zeros((B, S_ROWS - H0 * W0, C_IN), bf16)

    def conv(src_load, w_ref, b_ref, K, w_in, n_rows):
        acc = jnp.zeros((B * n_rows, C), f32) + b_ref[...]
        for dy in range(K):
            for dx in range(K):
                lhs = src_load(dy * w_in + dx, n_rows).reshape(B * n_rows, -1)
                acc = acc + jnp.dot(lhs, w_ref[dy * K + dx],
                                    preferred_element_type=f32)
        return _leaky(acc)

    act = conv(lambda off, n: s_scr[:, off:off + n, :],
               w1_ref, b1_ref, K=5, w_in=W0, n_rows=H1 * W0)
    wide_ref[...] = act.astype(bf16).reshape(B, H1 * W0, C)
    a1_ref[:, H1 * W1:, :] = jnp.zeros((B, A1_ROWS - H1 * W1, C), bf16)
    for y in range(H1):
        a1_ref[:, y * W1:(y + 1) * W1, :] = wide_ref[:, y * W0:y * W0 + W1, :]

    act = conv(lambda off, n: a1_ref[:, off:off + n, :],
               w2_ref, b2_ref, K=3, w_in=W1, n_rows=H2 * W1)
    wide_ref[:, :H2 * W1, :] = act.astype(bf16).reshape(B, H2 * W1, C)
    a2_ref[:, H2 * W2:, :] = jnp.zeros((B, A2_ROWS - H2 * W2, C), bf16)
    for y in range(H2):
        a2_ref[:, y * W2:(y + 1) * W2, :] = wide_ref[:, y * W1:y * W1 + W2, :]

    act = conv(lambda off, n: a2_ref[:, off:off + n, :],
               w3_ref, b3_ref, K=3, w_in=W2, n_rows=H3 * W2)
    act = act.reshape(B, H3 * W2, C)
    for y in range(H3):
        a3_ref[:, y * W3:(y + 1) * W3, :] = act[:, y * W2:y * W2 + W3, :]

    s = a3_ref[...].reshape(B * H3 * W3, C)
    xv = x_ref[...].reshape(B * H3 * W3, C)
    s = _leaky(jnp.dot(s, w4_ref[...], preferred_element_type=f32) + b4_ref[...])
    s = _leaky(jnp.dot(s, w5_ref[...], preferred_element_type=f32) + b5_ref[...])
    for m in range(4):
        s = _gelu(jnp.dot(s, wms_ref[m], preferred_element_type=f32)
                  + jnp.dot(xv, wmx_ref[m], preferred_element_type=f32)
                  + bm_ref[m])
    p = (jnp.dot(s, wcs_ref[...], preferred_element_type=f32)
         + jnp.dot(xv, wcx_ref[...], preferred_element_type=f32)
         + bc_ref[...])
    out_ref[...] = p.reshape(B, H3 * W3, 2)


def _features_head(s_flat, x_flat, p, B):
    N = s_flat.shape[0]
    full2 = lambda n: (0, 0)
    full3 = lambda n: (0, 0, 0)
    bf16 = jnp.bfloat16
    return pl.pallas_call(
        _feat_kernel,
        out_shape=jax.ShapeDtypeStruct((N, H3 * W3, 2), jnp.float32),
        grid_spec=pltpu.PrefetchScalarGridSpec(
            num_scalar_prefetch=0,
            grid=(N // B,),
            in_specs=[
                pl.BlockSpec((B, C_IN, H0 * W0), lambda n: (n, 0, 0)),
                pl.BlockSpec((B, H3 * W3, C), lambda n: (n, 0, 0)),
                pl.BlockSpec((25, C_IN, C), full3),
                pl.BlockSpec((1, C), full2),
                pl.BlockSpec((9, C, C), full3),
                pl.BlockSpec((1, C), full2),
                pl.BlockSpec((9, C, C), full3),
                pl.BlockSpec((1, C), full2),
                pl.BlockSpec((C, C), full2),
                pl.BlockSpec((1, C), full2),
                pl.BlockSpec((C, C), full2),
                pl.BlockSpec((1, C), full2),
                pl.BlockSpec((4, C, C), full3),
                pl.BlockSpec((4, C, C), full3),
                pl.BlockSpec((4, 1, C), full3),
                pl.BlockSpec((C, 2), full2),
                pl.BlockSpec((C, 2), full2),
                pl.BlockSpec((1, 2), full2),
            ],
            out_specs=pl.BlockSpec((B, H3 * W3, 2), lambda n: (n, 0, 0)),
            scratch_shapes=[
                pltpu.VMEM((B, S_ROWS, C_IN), bf16),
                pltpu.VMEM((B, H1 * W0, C), bf16),
                pltpu.VMEM((B, A1_ROWS, C), bf16),
                pltpu.VMEM((B, A2_ROWS, C), bf16),
                pltpu.VMEM((B, H3 * W3, C), jnp.float32),
            ],
        ),
        compiler_params=pltpu.CompilerParams(
            dimension_semantics=("parallel",),
            vmem_limit_bytes=64 * 1024 * 1024),
    )(s_flat, x_flat, p['w1'], p['b1'], p['w2'], p['b2'], p['w3'], p['b3'],
      p['w4'], p['b4'], p['w5'], p['b5'], p['wms'], p['wmx'], p['bm'],
      p['wcs'], p['wcx'], p['bc'])


def _interp_matrix(in_size, out_size):
    scale = in_size / out_size
    i = jnp.arange(out_size, dtype=jnp.float32)
    src = jnp.maximum((i + 0.5) * scale - 0.5, 0.0)
    i0 = jnp.minimum(jnp.floor(src).astype(jnp.int32), in_size - 1)
    i1 = jnp.minimum(i0 + 1, in_size - 1)
    w1 = src - i0.astype(jnp.float32)
    j = jnp.arange(in_size, dtype=jnp.int32)[None, :]
    return (jnp.where(j == i0[:, None], (1.0 - w1)[:, None], 0.0)
            + jnp.where(j == i1[:, None], w1[:, None], 0.0))


def _resize_kernel(q_ref, rh_ref, g_ref, out_ref):
    f32 = jnp.float32
    rb = q_ref.shape[0]
    rh = rh_ref[...]
    g = g_ref[...]
    for n in range(rb):
        t1 = jnp.dot(rh, q_ref[n], preferred_element_type=f32)
        both = jnp.dot(t1, g, preferred_element_type=f32)
        out_ref[n, 0] = both[:, :OUT_W]
        out_ref[n, 1] = both[:, OUT_W:]


def _bilinear_resize(p_flat):
    N = p_flat.shape[0]
    rh = _interp_matrix(H3, OUT_H)
    rwT = _interp_matrix(W3, OUT_W).T
    g = (rwT[:, None, None, :] * jnp.eye(2, dtype=jnp.float32)[None, :, :, None]
         ).reshape(2 * W3, 2 * OUT_W)
    q = p_flat.reshape(N, H3, W3 * 2)
    RB = 2
    while N % RB:
        RB //= 2
    return pl.pallas_call(
        _resize_kernel,
        out_shape=jax.ShapeDtypeStruct((N, 2, OUT_H, OUT_W), jnp.float32),
        grid_spec=pltpu.PrefetchScalarGridSpec(
            num_scalar_prefetch=0,
            grid=(N // RB,),
            in_specs=[
                pl.BlockSpec((RB, H3, W3 * 2), lambda n: (n, 0, 0)),
                pl.BlockSpec((OUT_H, H3), lambda n: (0, 0)),
                pl.BlockSpec((2 * W3, 2 * OUT_W), lambda n: (0, 0)),
            ],
            out_specs=pl.BlockSpec((RB, 2, OUT_H, OUT_W),
                                   lambda n: (n, 0, 0, 0)),
        ),
        compiler_params=pltpu.CompilerParams(
            dimension_semantics=("parallel",),
            vmem_limit_bytes=48 * 1024 * 1024),
    )(q, rh, g)


def kernel(x_nchw, s_nchw, w1, b1, w2, b2, w3, b3, w4, b4, w5, b5,
           wms, wmx, bm, wcs, wcx, bc):
    N = x_nchw.shape[0]
    B = 16
    while N % B:
        B //= 2

    bf16 = jnp.bfloat16
    s = s_nchw.reshape(N, C_IN, H0 * W0)
    x = jnp.transpose(x_nchw, (0, 2, 3, 1)).reshape(N, H3 * W3, C)

    p = {
        'w1': w1.astype(bf16), 'b1': b1, 'w2': w2.astype(bf16), 'b2': b2,
        'w3': w3.astype(bf16), 'b3': b3, 'w4': w4, 'b4': b4, 'w5': w5,
        'b5': b5, 'wms': wms, 'wmx': wmx, 'bm': bm, 'wcs': wcs,
        'wcx': wcx, 'bc': bc,
    }
    p_flat = _features_head(s, x, p, B)
    return _bilinear_resize(p_flat)


# speedup vs baseline: 1.0321x; 1.0321x over previous
import jax
import jax.numpy as jnp
from jax import lax
from jax.experimental import pallas as pl
from jax.experimental.pallas import tpu as pltpu

H0, W0, C_IN = 16, 16, 200
H1, W1 = 12, 12
H2, W2 = 10, 10
H3, W3 = 8, 8
C = 160
OUT_H, OUT_W = 512, 512

S_ROWS = 264
A1_ROWS = 152
A2_ROWS = 104


def _leaky(a, slope=0.2):
    return jnp.maximum(a, slope * a)


def _gelu(a):
    return 0.5 * a * (1.0 + lax.erf(a * 0.7071067811865476))


def _feat_kernel(s_ref, x_ref,
                 w1_ref, b1_ref, w2_ref, b2_ref, w3_ref, b3_ref,
                 w4_ref, b4_ref, w5_ref, b5_ref,
                 wms_ref, wmx_ref, bm_ref, wcs_ref, wcx_ref, bc_ref,
                 out_ref,
                 s_scr, wide_ref, a1_ref, a2_ref, a3_ref):
    f32 = jnp.float32
    bf16 = jnp.bfloat16
    B = s_ref.shape[0]

    s_scr[:, :H0 * W0, :] = jnp.swapaxes(s_ref[...].astype(bf16), 1, 2)
    s_scr[:, H0 * W0:, :] = jnp.zeros((B, S_ROWS - H0 * W0, C_IN), bf16)

    def conv(src_load, w_ref, b_ref, K, w_in, n_rows):
        acc = jnp.zeros((B * n_rows, C), f32) + b_ref[...]
        for dy in range(K):
            for dx in range(K):
                lhs = src_load(dy * w_in + dx, n_rows).reshape(B * n_rows, -1)
                acc = acc + jnp.dot(lhs, w_ref[dy * K + dx],
                                    preferred_element_type=f32)
        return _leaky(acc)

    act = conv(lambda off, n: s_scr[:, off:off + n, :],
               w1_ref, b1_ref, K=5, w_in=W0, n_rows=H1 * W0)
    wide_ref[...] = act.astype(bf16).reshape(B, H1 * W0, C)
    a1_ref[:, H1 * W1:, :] = jnp.zeros((B, A1_ROWS - H1 * W1, C), bf16)
    for y in range(H1):
        a1_ref[:, y * W1:(y + 1) * W1, :] = wide_ref[:, y * W0:y * W0 + W1, :]

    act = conv(lambda off, n: a1_ref[:, off:off + n, :],
               w2_ref, b2_ref, K=3, w_in=W1, n_rows=H2 * W1)
    wide_ref[:, :H2 * W1, :] = act.astype(bf16).reshape(B, H2 * W1, C)
    a2_ref[:, H2 * W2:, :] = jnp.zeros((B, A2_ROWS - H2 * W2, C), bf16)
    for y in range(H2):
        a2_ref[:, y * W2:(y + 1) * W2, :] = wide_ref[:, y * W1:y * W1 + W2, :]

    act = conv(lambda off, n: a2_ref[:, off:off + n, :],
               w3_ref, b3_ref, K=3, w_in=W2, n_rows=H3 * W2)
    act = act.reshape(B, H3 * W2, C)
    for y in range(H3):
        a3_ref[:, y * W3:(y + 1) * W3, :] = act[:, y * W2:y * W2 + W3, :]

    s = a3_ref[...].reshape(B * H3 * W3, C)
    xv = x_ref[...].reshape(B * H3 * W3, C)
    s = _leaky(jnp.dot(s, w4_ref[...], preferred_element_type=f32) + b4_ref[...])
    s = _leaky(jnp.dot(s, w5_ref[...], preferred_element_type=f32) + b5_ref[...])
    for m in range(4):
        s = _gelu(jnp.dot(s, wms_ref[m], preferred_element_type=f32)
                  + jnp.dot(xv, wmx_ref[m], preferred_element_type=f32)
                  + bm_ref[m])
    p = (jnp.dot(s, wcs_ref[...], preferred_element_type=f32)
         + jnp.dot(xv, wcx_ref[...], preferred_element_type=f32)
         + bc_ref[...])
    out_ref[...] = p.reshape(B, H3 * W3, 2)


def _features_head(s_flat, x_flat, p, B):
    N = s_flat.shape[0]
    full2 = lambda n: (0, 0)
    full3 = lambda n: (0, 0, 0)
    bf16 = jnp.bfloat16
    return pl.pallas_call(
        _feat_kernel,
        out_shape=jax.ShapeDtypeStruct((N, H3 * W3, 2), jnp.float32),
        grid_spec=pltpu.PrefetchScalarGridSpec(
            num_scalar_prefetch=0,
            grid=(N // B,),
            in_specs=[
                pl.BlockSpec((B, C_IN, H0 * W0), lambda n: (n, 0, 0)),
                pl.BlockSpec((B, H3 * W3, C), lambda n: (n, 0, 0)),
                pl.BlockSpec((25, C_IN, C), full3),
                pl.BlockSpec((1, C), full2),
                pl.BlockSpec((9, C, C), full3),
                pl.BlockSpec((1, C), full2),
                pl.BlockSpec((9, C, C), full3),
                pl.BlockSpec((1, C), full2),
                pl.BlockSpec((C, C), full2),
                pl.BlockSpec((1, C), full2),
                pl.BlockSpec((C, C), full2),
                pl.BlockSpec((1, C), full2),
                pl.BlockSpec((4, C, C), full3),
                pl.BlockSpec((4, C, C), full3),
                pl.BlockSpec((4, 1, C), full3),
                pl.BlockSpec((C, 2), full2),
                pl.BlockSpec((C, 2), full2),
                pl.BlockSpec((1, 2), full2),
            ],
            out_specs=pl.BlockSpec((B, H3 * W3, 2), lambda n: (n, 0, 0)),
            scratch_shapes=[
                pltpu.VMEM((B, S_ROWS, C_IN), bf16),
                pltpu.VMEM((B, H1 * W0, C), bf16),
                pltpu.VMEM((B, A1_ROWS, C), bf16),
                pltpu.VMEM((B, A2_ROWS, C), bf16),
                pltpu.VMEM((B, H3 * W3, C), jnp.float32),
            ],
        ),
        compiler_params=pltpu.CompilerParams(
            dimension_semantics=("parallel",),
            vmem_limit_bytes=64 * 1024 * 1024),
    )(s_flat, x_flat, p['w1'], p['b1'], p['w2'], p['b2'], p['w3'], p['b3'],
      p['w4'], p['b4'], p['w5'], p['b5'], p['wms'], p['wmx'], p['bm'],
      p['wcs'], p['wcx'], p['bc'])


def _interp_matrix(in_size, out_size):
    scale = in_size / out_size
    i = jnp.arange(out_size, dtype=jnp.float32)
    src = jnp.maximum((i + 0.5) * scale - 0.5, 0.0)
    i0 = jnp.minimum(jnp.floor(src).astype(jnp.int32), in_size - 1)
    i1 = jnp.minimum(i0 + 1, in_size - 1)
    w1 = src - i0.astype(jnp.float32)
    j = jnp.arange(in_size, dtype=jnp.int32)[None, :]
    return (jnp.where(j == i0[:, None], (1.0 - w1)[:, None], 0.0)
            + jnp.where(j == i1[:, None], w1[:, None], 0.0))


def _resize_kernel(q_ref, rh_ref, g_ref, out_ref):
    f32 = jnp.float32
    rb = q_ref.shape[0]
    rh = rh_ref[...]
    g = g_ref[...]
    for n in range(rb):
        t1 = jnp.dot(rh, q_ref[n], preferred_element_type=f32)
        both = jnp.dot(t1, g, preferred_element_type=f32)
        out_ref[n, 0] = both[:, :OUT_W]
        out_ref[n, 1] = both[:, OUT_W:]


def _bilinear_resize(p_flat):
    N = p_flat.shape[0]
    rh = _interp_matrix(H3, OUT_H)
    rwT = _interp_matrix(W3, OUT_W).T
    g = (rwT[:, None, None, :] * jnp.eye(2, dtype=jnp.float32)[None, :, :, None]
         ).reshape(2 * W3, 2 * OUT_W)
    q = p_flat.reshape(N, H3, W3 * 2)
    RB = 4
    while N % RB:
        RB //= 2
    return pl.pallas_call(
        _resize_kernel,
        out_shape=jax.ShapeDtypeStruct((N, 2, OUT_H, OUT_W), jnp.float32),
        grid_spec=pltpu.PrefetchScalarGridSpec(
            num_scalar_prefetch=0,
            grid=(N // RB,),
            in_specs=[
                pl.BlockSpec((RB, H3, W3 * 2), lambda n: (n, 0, 0)),
                pl.BlockSpec((OUT_H, H3), lambda n: (0, 0)),
                pl.BlockSpec((2 * W3, 2 * OUT_W), lambda n: (0, 0)),
            ],
            out_specs=pl.BlockSpec((RB, 2, OUT_H, OUT_W),
                                   lambda n: (n, 0, 0, 0)),
        ),
        compiler_params=pltpu.CompilerParams(
            dimension_semantics=("parallel",),
            vmem_limit_bytes=48 * 1024 * 1024),
    )(q, rh, g)


def kernel(x_nchw, s_nchw, w1, b1, w2, b2, w3, b3, w4, b4, w5, b5,
           wms, wmx, bm, wcs, wcx, bc):
    N = x_nchw.shape[0]
    B = 16
    while N % B:
        B //= 2

    bf16 = jnp.bfloat16
    s = s_nchw.reshape(N, C_IN, H0 * W0)
    x = jnp.transpose(x_nchw, (0, 2, 3, 1)).reshape(N, H3 * W3, C)

    p = {
        'w1': w1.astype(bf16), 'b1': b1, 'w2': w2.astype(bf16), 'b2': b2,
        'w3': w3.astype(bf16), 'b3': b3, 'w4': w4, 'b4': b4, 'w5': w5,
        'b5': b5, 'wms': wms, 'wmx': wmx, 'bm': bm, 'wcs': wcs,
        'wcx': wcx, 'bc': bc,
    }
    p_flat = _features_head(s, x, p, B)
    return _bilinear_resize(p_flat)
